# baseline (speedup 1.0000x reference)
import functools
import math

import jax
import jax.numpy as jnp
from jax import lax
from jax.experimental import pallas as pl
from jax.experimental.pallas import tpu as pltpu
from jax.experimental.pallas import tpu_sc as plsc

_N = 10000
_E = 320000
_D = 128
_OUT = 128
_T = 3
_H = 8
_DK = _OUT // _H

_TI_W = 208
_TJ_W = 192
_RC_W = 224
_ACC_W = 144

_NB = 400
_C = 25
_NW = 32
_EPW = _E // _NW
_NCH = _EPW // _C
_K = 20
_NBLK = _NCH // _K
_RPT = _N // 16


def _tables_body(x_ref, oh_ref, h_ref, ws_ref, bs_ref, wn_ref, bn_ref,
                 whs_ref, bhs_ref, whn_ref, bhn_ref, ti_ref, tj_ref):
    x = x_ref[...]
    oh = oh_ref[...]
    q = oh @ bs_ref[...]
    k = oh @ bn_ref[...]
    for t in range(_T):
        xt = x * oh[:, t][:, None]
        q += lax.dot(xt, ws_ref[t], preferred_element_type=jnp.float32)
        k += lax.dot(xt, wn_ref[t], preferred_element_type=jnp.float32)
    h = h_ref[...]
    hs = lax.dot(h, whs_ref[...], preferred_element_type=jnp.float32) + bhs_ref[...]
    hn = lax.dot(h, whn_ref[...], preferred_element_type=jnp.float32) + bhn_ref[...]
    tcol = oh @ lax.broadcasted_iota(jnp.int32, (_T, 16), 0).astype(jnp.float32)
    ti_ref[...] = jnp.concatenate([q, hs, tcol], axis=1)
    tj_ref[...] = jnp.concatenate([k, hn], axis=1)


def _build_tables(node_inp, onehot, h_mat, W_sub, b_sub, W_neigh, b_neigh,
                  W_hsub, b_hsub, W_hneigh, b_hneigh):
    full = lambda shape: pl.BlockSpec(shape, lambda i: tuple(0 for _ in shape))
    return pl.pallas_call(
        _tables_body,
        grid=(_N // _NB,),
        in_specs=[
            pl.BlockSpec((_NB, _D), lambda i: (i, 0)),
            pl.BlockSpec((_NB, _T), lambda i: (i, 0)),
            pl.BlockSpec((_NB, 8), lambda i: (i, 0)),
            full((_T, _D, _OUT)), full((_T, _OUT)),
            full((_T, _D, _OUT)), full((_T, _OUT)),
            full((8, 64)), full((1, 64)),
            full((8, 64)), full((1, 64)),
        ],
        out_specs=[
            pl.BlockSpec((_NB, _TI_W), lambda i: (i, 0)),
            pl.BlockSpec((_NB, _TJ_W), lambda i: (i, 0)),
        ],
        out_shape=[
            jax.ShapeDtypeStruct((_N, _TI_W), jnp.float32),
            jax.ShapeDtypeStruct((_N, _TJ_W), jnp.float32),
        ],
    )(node_inp, onehot, h_mat, W_sub, b_sub, W_neigh, b_neigh,
      W_hsub, b_hsub.reshape(1, 64), W_hneigh, b_hneigh.reshape(1, 64))


def _edge_body(ei_hbm, ej_hbm, ti_hbm, tj_hbm, rc_hbm, zeros_hbm, out_hbm,
               eib_v, ejb_v, ti_a, ti_b, tj_a, tj_b, msg_a, msg_b, rc_v,
               acc_sh, gat_a, gat_b, sct_a, sct_b):
    cid = lax.axis_index("c")
    sid = lax.axis_index("s")
    wid = sid * 2 + cid

    pltpu.sync_copy(zeros_hbm, acc_sh.at[pl.ds(sid * _RPT, _RPT)])
    pltpu.sync_copy(rc_hbm, rc_v)
    plsc.subcore_barrier()

    lane = lax.iota(jnp.int32, 16)
    lo8 = lane < 8

    def edge_one(ti_v, tj_v, msg_v):
        def one(e):
            t_i = ti_v[e, pl.ds(192, 16)][0].astype(jnp.int32)
            acch = jnp.zeros((16,), jnp.float32)
            for v in range(4):
                hs = ti_v[e, pl.ds(128 + 16 * v, 16)]
                hn = tj_v[e, pl.ds(128 + 16 * v, 16)]
                rha2 = rc_v[t_i, pl.ds(128 + 16 * v, 16)]
                t = jnp.exp(hs * hn)
                acch = acch + rha2 / (t + 1.0)
            s2v = rc_v[t_i, pl.ds(208, 16)] - (acch + lax.rev(acch, (0,)))
            accq = jnp.zeros((16,), jnp.float32)
            ks = []
            for h in range(_H):
                q = ti_v[e, pl.ds(16 * h, 16)]
                k = tj_v[e, pl.ds(16 * h, 16)]
                ra2 = rc_v[t_i, pl.ds(16 * h, 16)]
                t = jnp.exp(q * k)
                accq = accq + ra2 / (t + 1.0)
                ks.append(k)
            s1v = rc_v[t_i, pl.ds(192, 16)] - (accq + lax.rev(accq, (0,)))
            exv = jnp.exp(jnp.clip(s1v * s2v, -60.0, 60.0))
            e0 = jnp.where(lo8, exv, 0.0)
            exd = e0 + lax.rev(e0, (0,))
            for h in range(_H):
                msg_v[e, pl.ds(16 * h, 16)] = ks[h] * exd
            msg_v[e, pl.ds(128, 16)] = e0

        def body(u, _):
            one(2 * u)
            one(2 * u + 1)
            return 0
        lax.fori_loop(0, _C // 2, body, 0)
        one(_C - 1)

    def issue_gat(r, ti_x, tj_x, sem):
        pltpu.async_copy(ti_hbm.at[eib_v.at[r]], ti_x, sem)
        pltpu.async_copy(tj_hbm.at[ejb_v.at[r]], tj_x, sem)

    def wait_gat(ti_x, tj_x, sem):
        pltpu.make_async_copy(ti_hbm.at[pl.ds(0, _C)], ti_x, sem).wait()
        pltpu.make_async_copy(tj_hbm.at[pl.ds(0, _C)], tj_x, sem).wait()

    def issue_sct(r, msg_x, sem):
        pltpu.async_copy(msg_x, acc_sh.at[eib_v.at[r]], sem, add=True)

    def wait_sct(msg_x, sem):
        pltpu.make_async_copy(msg_x, acc_sh.at[pl.ds(0, _C)], sem).wait()

    def block(bi, _):
        @pl.when(bi > 0)
        def _():
            wait_sct(msg_a, sct_a)
            wait_sct(msg_b, sct_b)
        row0 = wid * _NCH + bi * _K
        pltpu.sync_copy(ei_hbm.at[pl.ds(row0, _K)], eib_v)
        pltpu.sync_copy(ej_hbm.at[pl.ds(row0, _K)], ejb_v)
        issue_gat(0, ti_a, tj_a, gat_a)
        wait_gat(ti_a, tj_a, gat_a)
        issue_gat(1, ti_b, tj_b, gat_b)
        edge_one(ti_a, tj_a, msg_a)
        issue_sct(0, msg_a, sct_a)
        wait_gat(ti_b, tj_b, gat_b)
        issue_gat(2, ti_a, tj_a, gat_a)
        edge_one(ti_b, tj_b, msg_b)
        issue_sct(1, msg_b, sct_b)

        def pair(u, _):
            t = 2 * u
            wait_gat(ti_a, tj_a, gat_a)
            issue_gat(t + 1, ti_b, tj_b, gat_b)
            wait_sct(msg_a, sct_a)
            edge_one(ti_a, tj_a, msg_a)
            issue_sct(t, msg_a, sct_a)
            wait_gat(ti_b, tj_b, gat_b)

            @pl.when(u < _K // 2 - 1)
            def _():
                issue_gat(t + 2, ti_a, tj_a, gat_a)
            wait_sct(msg_b, sct_b)
            edge_one(ti_b, tj_b, msg_b)
            issue_sct(t + 1, msg_b, sct_b)
            return 0

        lax.fori_loop(1, _K // 2, pair, 0)
        return 0

    lax.fori_loop(0, _NBLK, block, 0)
    wait_sct(msg_a, sct_a)
    wait_sct(msg_b, sct_b)
    plsc.subcore_barrier()
    pltpu.sync_copy(acc_sh.at[pl.ds(sid * _RPT, _RPT)],
                    out_hbm.at[cid, pl.ds(sid * _RPT, _RPT)])


def _edge_phase(ei, ej, ti, tj, rc):
    mesh = plsc.VectorSubcoreMesh(core_axis_name="c", subcore_axis_name="s",
                                  num_cores=2, num_subcores=16)
    zeros = jnp.zeros((_RPT, _ACC_W), jnp.float32)
    fn = pl.kernel(
        _edge_body,
        out_type=jax.ShapeDtypeStruct((2, _N, _ACC_W), jnp.float32),
        mesh=mesh,
        compiler_params=pltpu.CompilerParams(use_tc_tiling_on_sc=False,
                                             needs_layout_passes=False),
        scratch_types=[
            pltpu.VMEM((_K, _C), jnp.int32),
            pltpu.VMEM((_K, _C), jnp.int32),
            pltpu.VMEM((_C, _TI_W), jnp.float32),
            pltpu.VMEM((_C, _TI_W), jnp.float32),
            pltpu.VMEM((_C, _TJ_W), jnp.float32),
            pltpu.VMEM((_C, _TJ_W), jnp.float32),
            pltpu.VMEM((_C, _ACC_W), jnp.float32),
            pltpu.VMEM((_C, _ACC_W), jnp.float32),
            pltpu.VMEM((_T, _RC_W), jnp.float32),
            pltpu.VMEM_SHARED((_N, _ACC_W), jnp.float32),
            pltpu.SemaphoreType.DMA,
            pltpu.SemaphoreType.DMA,
            pltpu.SemaphoreType.DMA,
            pltpu.SemaphoreType.DMA,
        ],
    )
    return fn(ei.reshape(_E // _C, _C), ej.reshape(_E // _C, _C), ti, tj,
              rc, zeros)


def _finish_body(acc_ref, g_ref, b_ref, o_ref):
    a = acc_ref[0] + acc_ref[1]
    r = jax.lax.broadcasted_iota(jnp.int32, (_OUT, _OUT), 0)
    c = jax.lax.broadcasted_iota(jnp.int32, (_OUT, _OUT), 1)
    l = r % 16
    dk = 2 * (r // 16) + (l >= 8).astype(jnp.int32)
    h = jnp.where(l < 8, l, 15 - l)
    perm = (c == h * 16 + dk)
    num = lax.dot(a[:, :_OUT], perm.astype(jnp.float32),
                  preferred_element_type=jnp.float32)
    sel = (jax.lax.broadcasted_iota(jnp.int32, (_ACC_W, _OUT), 0)
           == _OUT + jax.lax.broadcasted_iota(jnp.int32, (_ACC_W, _OUT), 1) // _DK)
    den = lax.dot(a, sel.astype(jnp.float32), preferred_element_type=jnp.float32)
    x = num / (den + 1e-16)
    g = 0.5 * x * (1.0 + lax.erf(x * (1.0 / math.sqrt(2.0))))
    mu = jnp.mean(g, axis=-1, keepdims=True)
    var = jnp.mean((g - mu) ** 2, axis=-1, keepdims=True)
    o_ref[...] = (g - mu) / jnp.sqrt(var + 1e-5) * g_ref[...] + b_ref[...]


def _finish(acc, gamma, beta):
    return pl.pallas_call(
        _finish_body,
        grid=(_N // _NB,),
        in_specs=[
            pl.BlockSpec((2, _NB, _ACC_W), lambda i: (0, i, 0)),
            pl.BlockSpec((1, _OUT), lambda i: (0, 0)),
            pl.BlockSpec((1, _OUT), lambda i: (0, 0)),
        ],
        out_specs=pl.BlockSpec((_NB, _OUT), lambda i: (i, 0)),
        out_shape=jax.ShapeDtypeStruct((_N, _OUT), jnp.float32),
    )(acc, gamma.reshape(1, _OUT), beta.reshape(1, _OUT))


def _fold_perm(width):
    p = []
    for pos in range(_H * width):
        m, l = divmod(pos, 16)
        mn = 2 * m + (1 if l >= 8 else 0)
        h = l if l < 8 else 15 - l
        p.append(h * width + mn)
    return jnp.asarray(p, dtype=jnp.int32)


def kernel(node_inp, node_type, edge_index, h_mat, W_sub, b_sub, W_neigh,
           b_neigh, W_hsub, b_hsub, W_hneigh, b_hneigh, relation_att,
           relation_h_att, ln_gamma, ln_beta):
    pq = _fold_perm(_DK)
    ph = _fold_perm(8)
    onehot = (node_type[:, None] == jnp.arange(_T)[None, :]).astype(jnp.float32)
    ti, tj = _build_tables(
        node_inp, onehot, h_mat, 2.0 * W_sub[:, :, pq], 2.0 * b_sub[:, pq],
        W_neigh[:, :, pq], b_neigh[:, pq],
        2.0 * W_hsub[:, ph], 2.0 * b_hsub[ph], W_hneigh[:, ph], b_hneigh[ph])
    ra_s = relation_att.reshape(_T, _OUT) * (1.0 / math.sqrt(float(_DK)))
    rha_s = relation_h_att.reshape(_T, _H * 8) * (1.0 / math.sqrt(float(_H)))
    pad8 = jnp.zeros((_T, 8), jnp.float32)
    rc = jnp.concatenate(
        [2.0 * ra_s[:, pq], 2.0 * rha_s[:, ph],
         jnp.concatenate([ra_s.reshape(_T, _H, _DK).sum(-1), pad8], axis=1),
         jnp.concatenate([rha_s.reshape(_T, _H, 8).sum(-1), pad8], axis=1)],
        axis=1)
    ej = edge_index[0]
    ei = edge_index[1]
    acc = _edge_phase(ei, ej, ti, tj, rc)
    return _finish(acc, ln_gamma, ln_beta)


# speedup vs baseline: 1.7917x; 1.7917x over previous
import functools
import math

import jax
import jax.numpy as jnp
from jax import lax
from jax.experimental import pallas as pl
from jax.experimental.pallas import tpu as pltpu
from jax.experimental.pallas import tpu_sc as plsc

_N = 10000
_E = 320000
_D = 128
_OUT = 128
_T = 3
_H = 8
_DK = _OUT // _H

_TI_W = 208
_TJ_W = 192
_RC_W = 224
_ACC_W = 144

_NB = 400
_C = 25
_NW = 32
_EPW = _E // _NW
_NCH = _EPW // _C
_K = 20
_NBLK = _NCH // _K
_RPT = _N // 16


def _tables_body(x_ref, oh_ref, h_ref, ws_ref, bs_ref, wn_ref, bn_ref,
                 whs_ref, bhs_ref, whn_ref, bhn_ref, ti_ref, tj_ref):
    x = x_ref[...]
    oh = oh_ref[...]
    q = oh @ bs_ref[...]
    k = oh @ bn_ref[...]
    for t in range(_T):
        xt = x * oh[:, t][:, None]
        q += lax.dot(xt, ws_ref[t], preferred_element_type=jnp.float32)
        k += lax.dot(xt, wn_ref[t], preferred_element_type=jnp.float32)
    h = h_ref[...]
    hs = lax.dot(h, whs_ref[...], preferred_element_type=jnp.float32) + bhs_ref[...]
    hn = lax.dot(h, whn_ref[...], preferred_element_type=jnp.float32) + bhn_ref[...]
    tcol = oh @ lax.broadcasted_iota(jnp.int32, (_T, 16), 0).astype(jnp.float32)
    ti_ref[...] = jnp.concatenate([q, hs, tcol], axis=1)
    tj_ref[...] = jnp.concatenate([k, hn], axis=1)


def _build_tables(node_inp, onehot, h_mat, W_sub, b_sub, W_neigh, b_neigh,
                  W_hsub, b_hsub, W_hneigh, b_hneigh):
    full = lambda shape: pl.BlockSpec(shape, lambda i: tuple(0 for _ in shape))
    return pl.pallas_call(
        _tables_body,
        grid=(_N // _NB,),
        in_specs=[
            pl.BlockSpec((_NB, _D), lambda i: (i, 0)),
            pl.BlockSpec((_NB, _T), lambda i: (i, 0)),
            pl.BlockSpec((_NB, 8), lambda i: (i, 0)),
            full((_T, _D, _OUT)), full((_T, _OUT)),
            full((_T, _D, _OUT)), full((_T, _OUT)),
            full((8, 64)), full((1, 64)),
            full((8, 64)), full((1, 64)),
        ],
        out_specs=[
            pl.BlockSpec((_NB, _TI_W), lambda i: (i, 0)),
            pl.BlockSpec((_NB, _TJ_W), lambda i: (i, 0)),
        ],
        out_shape=[
            jax.ShapeDtypeStruct((_N, _TI_W), jnp.float32),
            jax.ShapeDtypeStruct((_N, _TJ_W), jnp.float32),
        ],
    )(node_inp, onehot, h_mat, W_sub, b_sub, W_neigh, b_neigh,
      W_hsub, b_hsub.reshape(1, 64), W_hneigh, b_hneigh.reshape(1, 64))


def _edge_body(ei_hbm, ej_hbm, ti_hbm, tj_hbm, rc_hbm, zeros_hbm, out_hbm,
               eib_v, ejb_v, ti_a, ti_b, tj_a, tj_b, msg_a, msg_b, rc_v,
               acc_sh, gat_a, gat_b, sct_a, sct_b):
    cid = lax.axis_index("c")
    sid = lax.axis_index("s")
    wid = sid * 2 + cid

    pltpu.sync_copy(zeros_hbm, acc_sh.at[pl.ds(sid * _RPT, _RPT)])
    pltpu.sync_copy(rc_hbm, rc_v)
    plsc.subcore_barrier()

    lane = lax.iota(jnp.int32, 16)
    lo8 = lane < 8

    def edge_one(ti_v, tj_v, msg_v):
        def one(e):
            t_i = ti_v[e, pl.ds(192, 16)][0].astype(jnp.int32)
            acch = jnp.zeros((16,), jnp.float32)
            for v in range(4):
                hs = ti_v[e, pl.ds(128 + 16 * v, 16)]
                hn = tj_v[e, pl.ds(128 + 16 * v, 16)]
                rha2 = rc_v[t_i, pl.ds(128 + 16 * v, 16)]
                t = jnp.exp(hs * hn)
                acch = acch + rha2 / (t + 1.0)
            s2v = rc_v[t_i, pl.ds(208, 16)] - (acch + lax.rev(acch, (0,)))
            accq = jnp.zeros((16,), jnp.float32)
            ks = []
            for h in range(_H):
                q = ti_v[e, pl.ds(16 * h, 16)]
                k = tj_v[e, pl.ds(16 * h, 16)]
                ra2 = rc_v[t_i, pl.ds(16 * h, 16)]
                t = jnp.exp(q * k)
                accq = accq + ra2 / (t + 1.0)
                ks.append(k)
            s1v = rc_v[t_i, pl.ds(192, 16)] - (accq + lax.rev(accq, (0,)))
            exv = jnp.exp(jnp.clip(s1v * s2v, -60.0, 60.0))
            e0 = jnp.where(lo8, exv, 0.0)
            exd = e0 + lax.rev(e0, (0,))
            for h in range(_H):
                msg_v[e, pl.ds(16 * h, 16)] = ks[h] * exd
            msg_v[e, pl.ds(128, 16)] = e0

        def body(e, _):
            one(e)
            return 0
        lax.fori_loop(0, _C, body, 0)

    def issue_gat(r, ti_x, tj_x, sem):
        pltpu.async_copy(ti_hbm.at[eib_v.at[r]], ti_x, sem)
        pltpu.async_copy(tj_hbm.at[ejb_v.at[r]], tj_x, sem)

    def wait_gat(ti_x, tj_x, sem):
        pltpu.make_async_copy(ti_hbm.at[pl.ds(0, _C)], ti_x, sem).wait()
        pltpu.make_async_copy(tj_hbm.at[pl.ds(0, _C)], tj_x, sem).wait()

    def issue_sct(r, msg_x, sem):
        pltpu.async_copy(msg_x, acc_sh.at[eib_v.at[r]], sem, add=True)

    def wait_sct(msg_x, sem):
        pltpu.make_async_copy(msg_x, acc_sh.at[pl.ds(0, _C)], sem).wait()

    def block(bi, _):
        @pl.when(bi > 0)
        def _():
            wait_sct(msg_a, sct_a)
            wait_sct(msg_b, sct_b)
        row0 = wid * _NCH + bi * _K
        pltpu.sync_copy(ei_hbm.at[pl.ds(row0, _K)], eib_v)
        pltpu.sync_copy(ej_hbm.at[pl.ds(row0, _K)], ejb_v)
        issue_gat(0, ti_a, tj_a, gat_a)
        wait_gat(ti_a, tj_a, gat_a)
        issue_gat(1, ti_b, tj_b, gat_b)
        edge_one(ti_a, tj_a, msg_a)
        issue_sct(0, msg_a, sct_a)
        wait_gat(ti_b, tj_b, gat_b)
        issue_gat(2, ti_a, tj_a, gat_a)
        edge_one(ti_b, tj_b, msg_b)
        issue_sct(1, msg_b, sct_b)

        def pair(u, _):
            t = 2 * u
            wait_gat(ti_a, tj_a, gat_a)
            issue_gat(t + 1, ti_b, tj_b, gat_b)
            wait_sct(msg_a, sct_a)
            edge_one(ti_a, tj_a, msg_a)
            issue_sct(t, msg_a, sct_a)
            wait_gat(ti_b, tj_b, gat_b)

            @pl.when(u < _K // 2 - 1)
            def _():
                issue_gat(t + 2, ti_a, tj_a, gat_a)
            wait_sct(msg_b, sct_b)
            edge_one(ti_b, tj_b, msg_b)
            issue_sct(t + 1, msg_b, sct_b)
            return 0

        lax.fori_loop(1, _K // 2, pair, 0)
        return 0

    lax.fori_loop(0, _NBLK, block, 0)
    wait_sct(msg_a, sct_a)
    wait_sct(msg_b, sct_b)
    plsc.subcore_barrier()
    pltpu.sync_copy(acc_sh.at[pl.ds(sid * _RPT, _RPT)],
                    out_hbm.at[cid, pl.ds(sid * _RPT, _RPT)])


def _edge_phase(ei, ej, ti, tj, rc):
    mesh = plsc.VectorSubcoreMesh(core_axis_name="c", subcore_axis_name="s",
                                  num_cores=2, num_subcores=16)
    zeros = jnp.zeros((_RPT, _ACC_W), jnp.float32)
    fn = pl.kernel(
        _edge_body,
        out_type=jax.ShapeDtypeStruct((2, _N, _ACC_W), jnp.float32),
        mesh=mesh,
        compiler_params=pltpu.CompilerParams(use_tc_tiling_on_sc=False,
                                             needs_layout_passes=False),
        scratch_types=[
            pltpu.VMEM((_K, _C), jnp.int32),
            pltpu.VMEM((_K, _C), jnp.int32),
            pltpu.VMEM((_C, _TI_W), jnp.float32),
            pltpu.VMEM((_C, _TI_W), jnp.float32),
            pltpu.VMEM((_C, _TJ_W), jnp.float32),
            pltpu.VMEM((_C, _TJ_W), jnp.float32),
            pltpu.VMEM((_C, _ACC_W), jnp.float32),
            pltpu.VMEM((_C, _ACC_W), jnp.float32),
            pltpu.VMEM((_T, _RC_W), jnp.float32),
            pltpu.VMEM_SHARED((_N, _ACC_W), jnp.float32),
            pltpu.SemaphoreType.DMA,
            pltpu.SemaphoreType.DMA,
            pltpu.SemaphoreType.DMA,
            pltpu.SemaphoreType.DMA,
        ],
    )
    return fn(ei.reshape(_E // _C, _C), ej.reshape(_E // _C, _C), ti, tj,
              rc, zeros)


def _finish_body(acc_ref, g_ref, b_ref, o_ref):
    a = acc_ref[0] + acc_ref[1]
    r = jax.lax.broadcasted_iota(jnp.int32, (_OUT, _OUT), 0)
    c = jax.lax.broadcasted_iota(jnp.int32, (_OUT, _OUT), 1)
    l = r % 16
    dk = 2 * (r // 16) + (l >= 8).astype(jnp.int32)
    h = jnp.where(l < 8, l, 15 - l)
    perm = (c == h * 16 + dk)
    num = lax.dot(a[:, :_OUT], perm.astype(jnp.float32),
                  preferred_element_type=jnp.float32)
    sel = (jax.lax.broadcasted_iota(jnp.int32, (_ACC_W, _OUT), 0)
           == _OUT + jax.lax.broadcasted_iota(jnp.int32, (_ACC_W, _OUT), 1) // _DK)
    den = lax.dot(a, sel.astype(jnp.float32), preferred_element_type=jnp.float32)
    x = num / (den + 1e-16)
    g = 0.5 * x * (1.0 + lax.erf(x * (1.0 / math.sqrt(2.0))))
    mu = jnp.mean(g, axis=-1, keepdims=True)
    var = jnp.mean((g - mu) ** 2, axis=-1, keepdims=True)
    o_ref[...] = (g - mu) / jnp.sqrt(var + 1e-5) * g_ref[...] + b_ref[...]


def _finish(acc, gamma, beta):
    return pl.pallas_call(
        _finish_body,
        grid=(_N // _NB,),
        in_specs=[
            pl.BlockSpec((2, _NB, _ACC_W), lambda i: (0, i, 0)),
            pl.BlockSpec((1, _OUT), lambda i: (0, 0)),
            pl.BlockSpec((1, _OUT), lambda i: (0, 0)),
        ],
        out_specs=pl.BlockSpec((_NB, _OUT), lambda i: (i, 0)),
        out_shape=jax.ShapeDtypeStruct((_N, _OUT), jnp.float32),
    )(acc, gamma.reshape(1, _OUT), beta.reshape(1, _OUT))


def _fold_perm(width):
    p = []
    for pos in range(_H * width):
        m, l = divmod(pos, 16)
        mn = 2 * m + (1 if l >= 8 else 0)
        h = l if l < 8 else 15 - l
        p.append(h * width + mn)
    return jnp.asarray(p, dtype=jnp.int32)


def kernel(node_inp, node_type, edge_index, h_mat, W_sub, b_sub, W_neigh,
           b_neigh, W_hsub, b_hsub, W_hneigh, b_hneigh, relation_att,
           relation_h_att, ln_gamma, ln_beta):
    pq = _fold_perm(_DK)
    ph = _fold_perm(8)
    onehot = (node_type[:, None] == jnp.arange(_T)[None, :]).astype(jnp.float32)
    ti, tj = _build_tables(
        node_inp, onehot, h_mat, 2.0 * W_sub[:, :, pq], 2.0 * b_sub[:, pq],
        W_neigh[:, :, pq], b_neigh[:, pq],
        2.0 * W_hsub[:, ph], 2.0 * b_hsub[ph], W_hneigh[:, ph], b_hneigh[ph])
    ra_s = relation_att.reshape(_T, _OUT) * (1.0 / math.sqrt(float(_DK)))
    rha_s = relation_h_att.reshape(_T, _H * 8) * (1.0 / math.sqrt(float(_H)))
    pad8 = jnp.zeros((_T, 8), jnp.float32)
    rc = jnp.concatenate(
        [2.0 * ra_s[:, pq], 2.0 * rha_s[:, ph],
         jnp.concatenate([ra_s.reshape(_T, _H, _DK).sum(-1), pad8], axis=1),
         jnp.concatenate([rha_s.reshape(_T, _H, 8).sum(-1), pad8], axis=1)],
        axis=1)
    ej = edge_index[0]
    ei = edge_index[1]
    acc = _edge_phase(ei, ej, ti, tj, rc)
    return _finish(acc, ln_gamma, ln_beta)


# speedup vs baseline: 2.2229x; 1.2406x over previous
import functools
import math

import jax
import jax.numpy as jnp
from jax import lax
from jax.experimental import pallas as pl
from jax.experimental.pallas import tpu as pltpu
from jax.experimental.pallas import tpu_sc as plsc

_N = 10000
_E = 320000
_D = 128
_OUT = 128
_T = 3
_H = 8
_DK = _OUT // _H

_TI_W = 208
_TJ_W = 192
_RC_W = 224
_ACC_W = 144

_NB = 400
_C = 25
_NW = 32
_EPW = _E // _NW
_NCH = _EPW // _C
_K = 20
_NBLK = _NCH // _K
_RPT = _N // 16


def _tables_body(x_ref, oh_ref, h_ref, ws_ref, bs_ref, wn_ref, bn_ref,
                 whs_ref, bhs_ref, whn_ref, bhn_ref, ti_ref, tj_ref):
    x = x_ref[...]
    oh = oh_ref[...]
    q = oh @ bs_ref[...]
    k = oh @ bn_ref[...]
    for t in range(_T):
        xt = x * oh[:, t][:, None]
        q += lax.dot(xt, ws_ref[t], preferred_element_type=jnp.float32)
        k += lax.dot(xt, wn_ref[t], preferred_element_type=jnp.float32)
    h = h_ref[...]
    hs = lax.dot(h, whs_ref[...], preferred_element_type=jnp.float32) + bhs_ref[...]
    hn = lax.dot(h, whn_ref[...], preferred_element_type=jnp.float32) + bhn_ref[...]
    tcol = oh @ lax.broadcasted_iota(jnp.int32, (_T, 16), 0).astype(jnp.float32)
    ti_ref[...] = jnp.concatenate([q, hs, tcol], axis=1)
    tj_ref[...] = jnp.concatenate([k, hn], axis=1)


def _build_tables(node_inp, onehot, h_mat, W_sub, b_sub, W_neigh, b_neigh,
                  W_hsub, b_hsub, W_hneigh, b_hneigh):
    full = lambda shape: pl.BlockSpec(shape, lambda i: tuple(0 for _ in shape))
    return pl.pallas_call(
        _tables_body,
        grid=(_N // _NB,),
        in_specs=[
            pl.BlockSpec((_NB, _D), lambda i: (i, 0)),
            pl.BlockSpec((_NB, _T), lambda i: (i, 0)),
            pl.BlockSpec((_NB, 8), lambda i: (i, 0)),
            full((_T, _D, _OUT)), full((_T, _OUT)),
            full((_T, _D, _OUT)), full((_T, _OUT)),
            full((8, 64)), full((1, 64)),
            full((8, 64)), full((1, 64)),
        ],
        out_specs=[
            pl.BlockSpec((_NB, _TI_W), lambda i: (i, 0)),
            pl.BlockSpec((_NB, _TJ_W), lambda i: (i, 0)),
        ],
        out_shape=[
            jax.ShapeDtypeStruct((_N, _TI_W), jnp.float32),
            jax.ShapeDtypeStruct((_N, _TJ_W), jnp.float32),
        ],
    )(node_inp, onehot, h_mat, W_sub, b_sub, W_neigh, b_neigh,
      W_hsub, b_hsub.reshape(1, 64), W_hneigh, b_hneigh.reshape(1, 64))


def _edge_body(ei_hbm, ej_hbm, ti_hbm, tj_hbm, rc_hbm, zeros_hbm, out_hbm,
               eib_v, ejb_v, ti_a, ti_b, tj_a, tj_b, msg_a, msg_b, rc_v,
               acc_sh, gat_a, gat_b, sct_a, sct_b):
    cid = lax.axis_index("c")
    sid = lax.axis_index("s")
    wid = sid * 2 + cid

    pltpu.sync_copy(zeros_hbm, acc_sh.at[pl.ds(sid * _RPT, _RPT)])
    pltpu.sync_copy(rc_hbm, rc_v)
    plsc.subcore_barrier()

    lane = lax.iota(jnp.int32, 16)
    lo8 = lane < 8

    def edge_one(ti_v, tj_v, msg_v):
        def one(e):
            t_i = ti_v[e, pl.ds(192, 16)][0].astype(jnp.int32)
            acch = jnp.zeros((16,), jnp.float32)
            for v in range(4):
                hs = ti_v[e, pl.ds(128 + 16 * v, 16)]
                hn = tj_v[e, pl.ds(128 + 16 * v, 16)]
                rha2 = rc_v[t_i, pl.ds(128 + 16 * v, 16)]
                t = jnp.exp(hs * hn)
                acch = acch + rha2 / (t + 1.0)
            s2v = rc_v[t_i, pl.ds(208, 16)] - (acch + lax.rev(acch, (0,)))
            accq = jnp.zeros((16,), jnp.float32)
            ks = []
            for h in range(_H):
                q = ti_v[e, pl.ds(16 * h, 16)]
                k = tj_v[e, pl.ds(16 * h, 16)]
                ra2 = rc_v[t_i, pl.ds(16 * h, 16)]
                t = jnp.exp(q * k)
                accq = accq + ra2 / (t + 1.0)
                ks.append(k)
            s1v = rc_v[t_i, pl.ds(192, 16)] - (accq + lax.rev(accq, (0,)))
            exv = jnp.exp(jnp.clip(s1v * s2v, -60.0, 60.0))
            e0 = jnp.where(lo8, exv, 0.0)
            exd = e0 + lax.rev(e0, (0,))
            for h in range(_H):
                msg_v[e, pl.ds(16 * h, 16)] = ks[h] * exd
            msg_v[e, pl.ds(128, 16)] = e0

        @functools.partial(plsc.parallel_loop, 0, _C, unroll=2)
        def body(e):
            one(e)

    def issue_gat(r, ti_x, tj_x, sem):
        pltpu.async_copy(ti_hbm.at[eib_v.at[r]], ti_x, sem)
        pltpu.async_copy(tj_hbm.at[ejb_v.at[r]], tj_x, sem)

    def wait_gat(ti_x, tj_x, sem):
        pltpu.make_async_copy(ti_hbm.at[pl.ds(0, _C)], ti_x, sem).wait()
        pltpu.make_async_copy(tj_hbm.at[pl.ds(0, _C)], tj_x, sem).wait()

    def issue_sct(r, msg_x, sem):
        pltpu.async_copy(msg_x, acc_sh.at[eib_v.at[r]], sem, add=True)

    def wait_sct(msg_x, sem):
        pltpu.make_async_copy(msg_x, acc_sh.at[pl.ds(0, _C)], sem).wait()

    def block(bi, _):
        @pl.when(bi > 0)
        def _():
            wait_sct(msg_a, sct_a)
            wait_sct(msg_b, sct_b)
        row0 = wid * _NCH + bi * _K
        pltpu.sync_copy(ei_hbm.at[pl.ds(row0, _K)], eib_v)
        pltpu.sync_copy(ej_hbm.at[pl.ds(row0, _K)], ejb_v)
        issue_gat(0, ti_a, tj_a, gat_a)
        wait_gat(ti_a, tj_a, gat_a)
        issue_gat(1, ti_b, tj_b, gat_b)
        edge_one(ti_a, tj_a, msg_a)
        issue_sct(0, msg_a, sct_a)
        wait_gat(ti_b, tj_b, gat_b)
        issue_gat(2, ti_a, tj_a, gat_a)
        edge_one(ti_b, tj_b, msg_b)
        issue_sct(1, msg_b, sct_b)

        def pair(u, _):
            t = 2 * u
            wait_gat(ti_a, tj_a, gat_a)
            issue_gat(t + 1, ti_b, tj_b, gat_b)
            wait_sct(msg_a, sct_a)
            edge_one(ti_a, tj_a, msg_a)
            issue_sct(t, msg_a, sct_a)
            wait_gat(ti_b, tj_b, gat_b)

            @pl.when(u < _K // 2 - 1)
            def _():
                issue_gat(t + 2, ti_a, tj_a, gat_a)
            wait_sct(msg_b, sct_b)
            edge_one(ti_b, tj_b, msg_b)
            issue_sct(t + 1, msg_b, sct_b)
            return 0

        lax.fori_loop(1, _K // 2, pair, 0)
        return 0

    lax.fori_loop(0, _NBLK, block, 0)
    wait_sct(msg_a, sct_a)
    wait_sct(msg_b, sct_b)
    plsc.subcore_barrier()
    pltpu.sync_copy(acc_sh.at[pl.ds(sid * _RPT, _RPT)],
                    out_hbm.at[cid, pl.ds(sid * _RPT, _RPT)])


def _edge_phase(ei, ej, ti, tj, rc):
    mesh = plsc.VectorSubcoreMesh(core_axis_name="c", subcore_axis_name="s",
                                  num_cores=2, num_subcores=16)
    zeros = jnp.zeros((_RPT, _ACC_W), jnp.float32)
    fn = pl.kernel(
        _edge_body,
        out_type=jax.ShapeDtypeStruct((2, _N, _ACC_W), jnp.float32),
        mesh=mesh,
        compiler_params=pltpu.CompilerParams(use_tc_tiling_on_sc=False,
                                             needs_layout_passes=False),
        scratch_types=[
            pltpu.VMEM((_K, _C), jnp.int32),
            pltpu.VMEM((_K, _C), jnp.int32),
            pltpu.VMEM((_C, _TI_W), jnp.float32),
            pltpu.VMEM((_C, _TI_W), jnp.float32),
            pltpu.VMEM((_C, _TJ_W), jnp.float32),
            pltpu.VMEM((_C, _TJ_W), jnp.float32),
            pltpu.VMEM((_C, _ACC_W), jnp.float32),
            pltpu.VMEM((_C, _ACC_W), jnp.float32),
            pltpu.VMEM((_T, _RC_W), jnp.float32),
            pltpu.VMEM_SHARED((_N, _ACC_W), jnp.float32),
            pltpu.SemaphoreType.DMA,
            pltpu.SemaphoreType.DMA,
            pltpu.SemaphoreType.DMA,
            pltpu.SemaphoreType.DMA,
        ],
    )
    return fn(ei.reshape(_E // _C, _C), ej.reshape(_E // _C, _C), ti, tj,
              rc, zeros)


def _finish_body(acc_ref, g_ref, b_ref, o_ref):
    a = acc_ref[0] + acc_ref[1]
    r = jax.lax.broadcasted_iota(jnp.int32, (_OUT, _OUT), 0)
    c = jax.lax.broadcasted_iota(jnp.int32, (_OUT, _OUT), 1)
    l = r % 16
    dk = 2 * (r // 16) + (l >= 8).astype(jnp.int32)
    h = jnp.where(l < 8, l, 15 - l)
    perm = (c == h * 16 + dk)
    num = lax.dot(a[:, :_OUT], perm.astype(jnp.float32),
                  preferred_element_type=jnp.float32)
    sel = (jax.lax.broadcasted_iota(jnp.int32, (_ACC_W, _OUT), 0)
           == _OUT + jax.lax.broadcasted_iota(jnp.int32, (_ACC_W, _OUT), 1) // _DK)
    den = lax.dot(a, sel.astype(jnp.float32), preferred_element_type=jnp.float32)
    x = num / (den + 1e-16)
    g = 0.5 * x * (1.0 + lax.erf(x * (1.0 / math.sqrt(2.0))))
    mu = jnp.mean(g, axis=-1, keepdims=True)
    var = jnp.mean((g - mu) ** 2, axis=-1, keepdims=True)
    o_ref[...] = (g - mu) / jnp.sqrt(var + 1e-5) * g_ref[...] + b_ref[...]


def _finish(acc, gamma, beta):
    return pl.pallas_call(
        _finish_body,
        grid=(_N // _NB,),
        in_specs=[
            pl.BlockSpec((2, _NB, _ACC_W), lambda i: (0, i, 0)),
            pl.BlockSpec((1, _OUT), lambda i: (0, 0)),
            pl.BlockSpec((1, _OUT), lambda i: (0, 0)),
        ],
        out_specs=pl.BlockSpec((_NB, _OUT), lambda i: (i, 0)),
        out_shape=jax.ShapeDtypeStruct((_N, _OUT), jnp.float32),
    )(acc, gamma.reshape(1, _OUT), beta.reshape(1, _OUT))


def _fold_perm(width):
    p = []
    for pos in range(_H * width):
        m, l = divmod(pos, 16)
        mn = 2 * m + (1 if l >= 8 else 0)
        h = l if l < 8 else 15 - l
        p.append(h * width + mn)
    return jnp.asarray(p, dtype=jnp.int32)


def kernel(node_inp, node_type, edge_index, h_mat, W_sub, b_sub, W_neigh,
           b_neigh, W_hsub, b_hsub, W_hneigh, b_hneigh, relation_att,
           relation_h_att, ln_gamma, ln_beta):
    pq = _fold_perm(_DK)
    ph = _fold_perm(8)
    onehot = (node_type[:, None] == jnp.arange(_T)[None, :]).astype(jnp.float32)
    ti, tj = _build_tables(
        node_inp, onehot, h_mat, 2.0 * W_sub[:, :, pq], 2.0 * b_sub[:, pq],
        W_neigh[:, :, pq], b_neigh[:, pq],
        2.0 * W_hsub[:, ph], 2.0 * b_hsub[ph], W_hneigh[:, ph], b_hneigh[ph])
    ra_s = relation_att.reshape(_T, _OUT) * (1.0 / math.sqrt(float(_DK)))
    rha_s = relation_h_att.reshape(_T, _H * 8) * (1.0 / math.sqrt(float(_H)))
    pad8 = jnp.zeros((_T, 8), jnp.float32)
    rc = jnp.concatenate(
        [2.0 * ra_s[:, pq], 2.0 * rha_s[:, ph],
         jnp.concatenate([ra_s.reshape(_T, _H, _DK).sum(-1), pad8], axis=1),
         jnp.concatenate([rha_s.reshape(_T, _H, 8).sum(-1), pad8], axis=1)],
        axis=1)
    ej = edge_index[0]
    ei = edge_index[1]
    acc = _edge_phase(ei, ej, ti, tj, rc)
    return _finish(acc, ln_gamma, ln_beta)
